# Initial kernel scaffold; baseline (speedup 1.0000x reference)
import jax
import jax.numpy as jnp
from jax.experimental import pallas as pl


def kernel(X, edge_index, edge_vals, W1, W2):
    raise NotImplementedError("write your pallas kernel here")


# speedup vs baseline: 4.2546x; 4.2546x over previous
import functools

import jax
import jax.numpy as jnp
from jax import lax
from jax.experimental import pallas as pl
from jax.experimental.pallas import tpu as pltpu
from jax.experimental.pallas import tpu_sc as plsc

N_NODES = 10000
N_EDGES = 320000
D = 128

NC = 2
NS = 16
EDGES_PER_SUB = N_EDGES // (NC * NS)
B = 80
NBATCH = EDGES_PER_SUB // B
ROWS_PER_SUB = N_NODES // NS

_MM_BM = 2000


def _mm_body(relu_out, pa_ref, pb_ref, w_ref, y_ref):
    h = pa_ref[...] + pb_ref[...]
    y = lax.dot_general(h, w_ref[...], (((1,), (1,)), ((), ())),
                        preferred_element_type=jnp.float32,
                        precision=lax.Precision.HIGHEST)
    if relu_out:
        y = jnp.maximum(y, 0.0)
    y_ref[...] = y


def _mm(pa, pb, w, relu_out):
    return pl.pallas_call(
        functools.partial(_mm_body, relu_out),
        grid=(N_NODES // _MM_BM,),
        in_specs=[
            pl.BlockSpec((_MM_BM, D), lambda i: (i, 0)),
            pl.BlockSpec((_MM_BM, D), lambda i: (i, 0)),
            pl.BlockSpec((D, D), lambda i: (0, 0)),
        ],
        out_specs=pl.BlockSpec((_MM_BM, D), lambda i: (i, 0)),
        out_shape=jax.ShapeDtypeStruct((N_NODES, D), jnp.float32),
    )(pa, pb, w)


def _spmm_kernel(row_hbm, col_hbm, val_hbm, y_hbm, outa_hbm, outb_hbm,
                 acc, rowbuf, colbuf, vbuf, gbuf, zbuf, sem):
    c = lax.axis_index("c")
    s = lax.axis_index("s")

    def zrow(i, _):
        for f in range(D // 16):
            zbuf[i, pl.ds(f * 16, 16)] = jnp.zeros((16,), jnp.float32)
        return 0
    lax.fori_loop(0, 125, zrow, 0)
    for t in range(ROWS_PER_SUB // 125):
        pltpu.sync_copy(zbuf, acc.at[pl.ds(s * ROWS_PER_SUB + t * 125, 125)])
    plsc.subcore_barrier()

    base = (c * NS + s) * EDGES_PER_SUB

    def batch(j, _):
        b0 = base + j * B
        pltpu.sync_copy(row_hbm.at[pl.ds(b0, B)], rowbuf.at[0])
        pltpu.sync_copy(col_hbm.at[pl.ds(b0, B)], colbuf)
        pltpu.sync_copy(val_hbm.at[pl.ds(b0, B)], vbuf)
        pltpu.async_copy(y_hbm.at[colbuf], gbuf, sem).wait()

        def grp(g, _):
            vv16 = vbuf[pl.ds(g * 16, 16)]
            for k in range(16):
                e = g * 16 + k
                vv = vv16[k]
                for f in range(D // 16):
                    sl = pl.ds(f * 16, 16)
                    gbuf[e, sl] = gbuf[e, sl] * vv
            return 0
        lax.fori_loop(0, B // 16, grp, 0)

        pltpu.sync_copy(gbuf, acc.at[rowbuf.at[0]], add=True)
        return 0
    lax.fori_loop(0, NBATCH, batch, 0)
    plsc.subcore_barrier()

    @pl.when(jnp.logical_and(c == 0, s < 10))
    def _():
        r0 = s * 1000
        pltpu.sync_copy(acc.at[pl.ds(r0, 1000)],
                        outa_hbm.at[pl.ds(r0, 1000)])

    @pl.when(jnp.logical_and(c == 1, s < 10))
    def _():
        r0 = s * 1000
        pltpu.sync_copy(acc.at[pl.ds(r0, 1000)],
                        outb_hbm.at[pl.ds(r0, 1000)])


_spmm = pl.kernel(
    _spmm_kernel,
    out_type=[jax.ShapeDtypeStruct((N_NODES, D), jnp.float32)] * 2,
    mesh=plsc.VectorSubcoreMesh(core_axis_name="c", subcore_axis_name="s"),
    scratch_types=[
        pltpu.VMEM_SHARED((N_NODES, D), jnp.float32),
        pltpu.VMEM((1, B), jnp.int32),
        pltpu.VMEM((B,), jnp.int32),
        pltpu.VMEM((B,), jnp.float32),
        pltpu.VMEM((B, D), jnp.float32),
        pltpu.VMEM((125, D), jnp.float32),
        pltpu.SemaphoreType.DMA,
    ],
)


def kernel(X, edge_index, edge_vals, W1, W2):
    row = edge_index[0].astype(jnp.int32)
    col = edge_index[1].astype(jnp.int32)
    vals = edge_vals.astype(jnp.float32)

    pa1, pb1 = _spmm(row, col, vals, X)
    h = _mm(pa1, pb1, W1, relu_out=True)
    pa2, pb2 = _spmm(row, col, vals, h)
    return _mm(pa2, pb2, W2, relu_out=False)


# speedup vs baseline: 10.2182x; 2.4017x over previous
import functools

import jax
import jax.numpy as jnp
from jax import lax
from jax.experimental import pallas as pl
from jax.experimental.pallas import tpu as pltpu
from jax.experimental.pallas import tpu_sc as plsc

N_NODES = 10000
N_EDGES = 320000
D = 128

NC = 2
NS = 16
EDGES_PER_SUB = N_EDGES // (NC * NS)
B = 80
NBATCH = EDGES_PER_SUB // B
SG = 5
BPG = NBATCH // SG
ROWS_PER_SUB = N_NODES // NS

_MM_BM = 2000


def _mm_body(relu_out, pa_ref, pb_ref, w_ref, y_ref):
    h = pa_ref[...] + pb_ref[...]
    y = lax.dot_general(h, w_ref[...], (((1,), (1,)), ((), ())),
                        preferred_element_type=jnp.float32,
                        precision=lax.Precision.HIGHEST)
    if relu_out:
        y = jnp.maximum(y, 0.0)
    y_ref[...] = y


def _mm(pa, pb, w, relu_out):
    return pl.pallas_call(
        functools.partial(_mm_body, relu_out),
        grid=(N_NODES // _MM_BM,),
        in_specs=[
            pl.BlockSpec((_MM_BM, D), lambda i: (i, 0)),
            pl.BlockSpec((_MM_BM, D), lambda i: (i, 0)),
            pl.BlockSpec((D, D), lambda i: (0, 0)),
        ],
        out_specs=pl.BlockSpec((_MM_BM, D), lambda i: (i, 0)),
        out_shape=jax.ShapeDtypeStruct((N_NODES, D), jnp.float32),
    )(pa, pb, w)


def _spmm_kernel(row_hbm, col_hbm, val_hbm, y_hbm, outa_hbm, outb_hbm,
                 acc, rowbuf, colbuf, vbuf, gbuf0, gbuf1, sem0, sem1):
    c = lax.axis_index("c")
    s = lax.axis_index("s")
    wid = c * NS + s

    def zrow(i, _):
        for f in range(D // 16):
            gbuf0[i, pl.ds(f * 16, 16)] = jnp.zeros((16,), jnp.float32)
        return 0
    lax.fori_loop(0, B, zrow, 0)
    r0 = s * ROWS_PER_SUB
    for t in range(ROWS_PER_SUB // B):
        pltpu.sync_copy(gbuf0, acc.at[pl.ds(r0 + t * B, B)])
    rem = ROWS_PER_SUB % B
    pltpu.sync_copy(gbuf0.at[pl.ds(0, rem)],
                    acc.at[pl.ds(r0 + (ROWS_PER_SUB // B) * B, rem)])
    plsc.subcore_barrier()

    def start(j, gslot, sem):
        pltpu.async_copy(y_hbm.at[colbuf.at[j]], gslot, sem)

    def finish(j, gslot, sem):
        pltpu.make_async_copy(y_hbm.at[colbuf.at[j]], gslot, sem).wait()

    def scale_scatter(j, gslot):
        def grp(g, _):
            vv16 = vbuf[j, pl.ds(g * 16, 16)]
            for k in range(16):
                e = g * 16 + k
                vv = vv16[k]
                for f in range(D // 16):
                    sl = pl.ds(f * 16, 16)
                    gslot[e, sl] = gslot[e, sl] * vv
            return 0
        lax.fori_loop(0, B // 16, grp, 0)
        pltpu.sync_copy(gslot, acc.at[rowbuf.at[j]], add=True)

    def group_fn(g, _):
        pltpu.sync_copy(row_hbm.at[wid, g], rowbuf)
        pltpu.sync_copy(col_hbm.at[wid, g], colbuf)
        pltpu.sync_copy(val_hbm.at[wid, g], vbuf)

        start(0, gbuf0, sem0)

        def body(i, _):
            j0 = 2 * i
            start(j0 + 1, gbuf1, sem1)
            finish(j0, gbuf0, sem0)
            scale_scatter(j0, gbuf0)
            start(j0 + 2, gbuf0, sem0)
            finish(j0 + 1, gbuf1, sem1)
            scale_scatter(j0 + 1, gbuf1)
            return 0
        lax.fori_loop(0, (BPG - 1) // 2, body, 0)
        finish(BPG - 1, gbuf0, sem0)
        scale_scatter(BPG - 1, gbuf0)
        return 0
    lax.fori_loop(0, SG, group_fn, 0)
    plsc.subcore_barrier()

    @pl.when(jnp.logical_and(c == 0, s < 10))
    def _():
        r0 = s * 1000
        pltpu.sync_copy(acc.at[pl.ds(r0, 1000)],
                        outa_hbm.at[pl.ds(r0, 1000)])

    @pl.when(jnp.logical_and(c == 1, s < 10))
    def _():
        r0 = s * 1000
        pltpu.sync_copy(acc.at[pl.ds(r0, 1000)],
                        outb_hbm.at[pl.ds(r0, 1000)])


_spmm = pl.kernel(
    _spmm_kernel,
    out_type=[jax.ShapeDtypeStruct((N_NODES, D), jnp.float32)] * 2,
    mesh=plsc.VectorSubcoreMesh(core_axis_name="c", subcore_axis_name="s"),
    scratch_types=[
        pltpu.VMEM_SHARED((N_NODES, D), jnp.float32),
        pltpu.VMEM((BPG, B), jnp.int32),
        pltpu.VMEM((BPG, B), jnp.int32),
        pltpu.VMEM((BPG, B), jnp.float32),
        pltpu.VMEM((B, D), jnp.float32),
        pltpu.VMEM((B, D), jnp.float32),
        pltpu.SemaphoreType.DMA,
        pltpu.SemaphoreType.DMA,
    ],
)


def kernel(X, edge_index, edge_vals, W1, W2):
    shape4 = (NC * NS, SG, BPG, B)
    row = edge_index[0].astype(jnp.int32).reshape(shape4)
    col = edge_index[1].astype(jnp.int32).reshape(shape4)
    vals = edge_vals.astype(jnp.float32).reshape(shape4)

    pa1, pb1 = _spmm(row, col, vals, X)
    h = _mm(pa1, pb1, W1, relu_out=True)
    pa2, pb2 = _spmm(row, col, vals, h)
    return _mm(pa2, pb2, W2, relu_out=False)


# speedup vs baseline: 11.2440x; 1.1004x over previous
import functools

import jax
import jax.numpy as jnp
from jax import lax
from jax.experimental import pallas as pl
from jax.experimental.pallas import tpu as pltpu
from jax.experimental.pallas import tpu_sc as plsc

N_NODES = 10000
N_EDGES = 320000
D = 128

NC = 2
NS = 16
EDGES_PER_SUB = N_EDGES // (NC * NS)
B = 80
NBATCH = EDGES_PER_SUB // B
SG = 5
BPG = NBATCH // SG
ROWS_PER_SUB = N_NODES // NS

_MM_BM = 2000


def _mm_body(relu_out, pa_ref, pb_ref, w_ref, y_ref):
    h = pa_ref[...] + pb_ref[...]
    y = lax.dot_general(h, w_ref[...], (((1,), (1,)), ((), ())),
                        preferred_element_type=jnp.float32,
                        precision=lax.Precision.HIGHEST)
    if relu_out:
        y = jnp.maximum(y, 0.0)
    y_ref[...] = y


def _mm(pa, pb, w, relu_out):
    return pl.pallas_call(
        functools.partial(_mm_body, relu_out),
        grid=(N_NODES // _MM_BM,),
        in_specs=[
            pl.BlockSpec((_MM_BM, D), lambda i: (i, 0)),
            pl.BlockSpec((_MM_BM, D), lambda i: (i, 0)),
            pl.BlockSpec((D, D), lambda i: (0, 0)),
        ],
        out_specs=pl.BlockSpec((_MM_BM, D), lambda i: (i, 0)),
        out_shape=jax.ShapeDtypeStruct((N_NODES, D), jnp.float32),
    )(pa, pb, w)


def _spmm_kernel(row_hbm, col_hbm, val_hbm, y_hbm, outa_hbm, outb_hbm,
                 acc, rowbuf, colbuf, vbuf, gbuf0, gbuf1, gbuf2,
                 sg0, sg1, sg2, ss0, ss1, ss2):
    c = lax.axis_index("c")
    s = lax.axis_index("s")
    wid = c * NS + s

    def zrow(i, _):
        for f in range(D // 16):
            gbuf0[i, pl.ds(f * 16, 16)] = jnp.zeros((16,), jnp.float32)
        return 0
    lax.fori_loop(0, B, zrow, 0)
    r0 = s * ROWS_PER_SUB
    for t in range(ROWS_PER_SUB // B):
        pltpu.sync_copy(gbuf0, acc.at[pl.ds(r0 + t * B, B)])
    rem = ROWS_PER_SUB % B
    pltpu.sync_copy(gbuf0.at[pl.ds(0, rem)],
                    acc.at[pl.ds(r0 + (ROWS_PER_SUB // B) * B, rem)])
    plsc.subcore_barrier()

    def gstart(j, buf, sem):
        pltpu.async_copy(y_hbm.at[colbuf.at[j]], buf, sem)

    def gwait(j, buf, sem):
        pltpu.make_async_copy(y_hbm.at[colbuf.at[j]], buf, sem).wait()

    def sstart(j, buf, sem):
        pltpu.async_copy(buf, acc.at[rowbuf.at[j]], sem, add=True)

    def swait(j, buf, sem):
        pltpu.make_async_copy(buf, acc.at[rowbuf.at[j]], sem).wait()

    def scale(j, buf):
        def grp(g, _):
            vv16 = vbuf[j, pl.ds(g * 16, 16)]
            for k in range(16):
                e = g * 16 + k
                vv = vv16[k]
                for f in range(D // 16):
                    sl = pl.ds(f * 16, 16)
                    buf[e, sl] = buf[e, sl] * vv
            return 0
        lax.fori_loop(0, B // 16, grp, 0)

    def step(j, buf, gsem, ssem, jp, pbuf, pgsem, psem):
        gwait(j, buf, gsem)
        scale(j, buf)
        sstart(j, buf, ssem)
        swait(jp, pbuf, psem)
        gstart(j + 2, pbuf, pgsem)

    def group_fn(g, _):
        pltpu.sync_copy(row_hbm.at[wid, g], rowbuf)
        pltpu.sync_copy(col_hbm.at[wid, g], colbuf)
        pltpu.sync_copy(val_hbm.at[wid, g], vbuf)

        gstart(0, gbuf0, sg0)
        gstart(1, gbuf1, sg1)
        gwait(0, gbuf0, sg0)
        scale(0, gbuf0)
        sstart(0, gbuf0, ss0)
        gstart(2, gbuf2, sg2)

        def sbody(i, _):
            j = 3 * i + 1
            step(j, gbuf1, sg1, ss1, j - 1, gbuf0, sg0, ss0)
            step(j + 1, gbuf2, sg2, ss2, j, gbuf1, sg1, ss1)
            step(j + 2, gbuf0, sg0, ss0, j + 1, gbuf2, sg2, ss2)
            return 0
        lax.fori_loop(0, (BPG - 4) // 3, sbody, 0)

        step(22, gbuf1, sg1, ss1, 21, gbuf0, sg0, ss0)
        gwait(23, gbuf2, sg2)
        scale(23, gbuf2)
        sstart(23, gbuf2, ss2)
        swait(22, gbuf1, ss1)
        gwait(24, gbuf0, sg0)
        scale(24, gbuf0)
        sstart(24, gbuf0, ss0)
        swait(23, gbuf2, ss2)
        swait(24, gbuf0, ss0)
        return 0
    lax.fori_loop(0, SG, group_fn, 0)
    plsc.subcore_barrier()

    @pl.when(jnp.logical_and(c == 0, s < 10))
    def _():
        r0 = s * 1000
        pltpu.sync_copy(acc.at[pl.ds(r0, 1000)],
                        outa_hbm.at[pl.ds(r0, 1000)])

    @pl.when(jnp.logical_and(c == 1, s < 10))
    def _():
        r0 = s * 1000
        pltpu.sync_copy(acc.at[pl.ds(r0, 1000)],
                        outb_hbm.at[pl.ds(r0, 1000)])


_spmm = pl.kernel(
    _spmm_kernel,
    out_type=[jax.ShapeDtypeStruct((N_NODES, D), jnp.float32)] * 2,
    mesh=plsc.VectorSubcoreMesh(core_axis_name="c", subcore_axis_name="s"),
    scratch_types=[
        pltpu.VMEM_SHARED((N_NODES, D), jnp.float32),
        pltpu.VMEM((BPG, B), jnp.int32),
        pltpu.VMEM((BPG, B), jnp.int32),
        pltpu.VMEM((BPG, B), jnp.float32),
        pltpu.VMEM((B, D), jnp.float32),
        pltpu.VMEM((B, D), jnp.float32),
        pltpu.VMEM((B, D), jnp.float32),
        pltpu.SemaphoreType.DMA,
        pltpu.SemaphoreType.DMA,
        pltpu.SemaphoreType.DMA,
        pltpu.SemaphoreType.DMA,
        pltpu.SemaphoreType.DMA,
        pltpu.SemaphoreType.DMA,
    ],
)


def kernel(X, edge_index, edge_vals, W1, W2):
    shape4 = (NC * NS, SG, BPG, B)
    row = edge_index[0].astype(jnp.int32).reshape(shape4)
    col = edge_index[1].astype(jnp.int32).reshape(shape4)
    vals = edge_vals.astype(jnp.float32).reshape(shape4)

    pa1, pb1 = _spmm(row, col, vals, X)
    h = _mm(pa1, pb1, W1, relu_out=True)
    pa2, pb2 = _spmm(row, col, vals, h)
    return _mm(pa2, pb2, W2, relu_out=False)
